# Initial kernel scaffold; baseline (speedup 1.0000x reference)
import jax
import jax.numpy as jnp
from jax.experimental import pallas as pl


def kernel(y, edge_index, edge_weight, W1_0, b1_0, W2_0, W3_0, b3_0, W1_1, b1_1, W2_1, W3_1, b3_1):
    raise NotImplementedError("write your pallas kernel here")


# speedup vs baseline: 5.4162x; 5.4162x over previous
import functools

import jax
import jax.numpy as jnp
from jax import lax
from jax.experimental import pallas as pl
from jax.experimental.pallas import tpu as pltpu
from jax.experimental.pallas import tpu_sc as plsc

N = 10000
E = 320000
D = 128
NC = 2
NS = 16
NW = NC * NS
EP = E // NW
K = 80
NCHUNK = EP // K
NPAD = 10240
ZROWS = NPAD // NS
DEGW = NPAD // NS
LG = D // 16

_GATHER_DN = lax.GatherDimensionNumbers(
    offset_dims=(), collapsed_slice_dims=(0,), start_index_map=(0,))


def _splat(v16, e):
    return lax.gather(v16, jnp.full((16, 1), e, jnp.int32), _GATHER_DN, (1,),
                      mode=lax.GatherScatterMode.PROMISE_IN_BOUNDS)


def _sc_body(compute_deg, a_hbm, src_hbm, dst_hbm, w_hbm, *rest):
    if compute_deg:
        (z_out, deg_out, zsh, dsh, srcv, dstv, wv, rows, zbuf,
         degloc, degtile, degres, gsem, ssem) = rest
    else:
        (z_out, zsh, srcv, dstv, wv, rows, zbuf, gsem, ssem) = rest
    c = lax.axis_index("c")
    s = lax.axis_index("s")
    wid = s * NC + c
    zero16 = jnp.zeros((16,), jnp.float32)

    def zero_zbuf(i, _):
        for j in range(LG):
            zbuf[i, pl.ds(j * 16, 16)] = zero16
        return 0
    lax.fori_loop(0, 32, zero_zbuf, 0)

    def zero_zsh(k, _):
        pltpu.sync_copy(zbuf, zsh.at[pl.ds(s * ZROWS + k * 32, 32)])
        return 0
    lax.fori_loop(0, ZROWS // 32, zero_zsh, 0)

    if compute_deg:
        def zero_deg(i, _):
            degloc[pl.ds(i * 16, 16)] = zero16
            return 0
        lax.fori_loop(0, NPAD // 16, zero_deg, 0)

    plsc.subcore_barrier()

    def chunk(i, _):
        off = pl.multiple_of(wid * EP + i * K, 8)
        pltpu.sync_copy(src_hbm.at[pl.ds(off, K)], srcv)
        pltpu.sync_copy(dst_hbm.at[pl.ds(off, K)], dstv)
        pltpu.sync_copy(w_hbm.at[pl.ds(off, K)], wv)
        pltpu.async_copy(a_hbm.at[srcv], rows, gsem).wait()
        for g in range(K // 16):
            w16 = wv[pl.ds(g * 16, 16)]
            if compute_deg:
                d16 = dstv[pl.ds(g * 16, 16)]
                plsc.addupdate_scatter(degloc, [d16], w16)
            for e in range(16):
                r = g * 16 + e
                we = _splat(w16, e)
                for j in range(LG):
                    rows[r, pl.ds(j * 16, 16)] = rows[r, pl.ds(j * 16, 16)] * we
        pltpu.async_copy(rows, zsh.at[dstv], ssem, add=True).wait()
        return 0
    lax.fori_loop(0, NCHUNK, chunk, 0)

    if compute_deg:
        pltpu.sync_copy(degloc, dsh.at[s])

    plsc.subcore_barrier()

    pltpu.sync_copy(zsh.at[pl.ds(s * ZROWS, ZROWS)],
                    z_out.at[c, pl.ds(s * ZROWS, ZROWS)])

    if compute_deg:
        pltpu.sync_copy(dsh.at[:, pl.ds(s * DEGW, DEGW)], degtile)

        def red(j, _):
            acc = zero16
            for r in range(NS):
                acc = acc + degtile[r, pl.ds(j * 16, 16)]
            degres[pl.ds(j * 16, 16)] = acc
            return 0
        lax.fori_loop(0, DEGW // 16, red, 0)
        pltpu.sync_copy(degres, deg_out.at[c, pl.ds(s * DEGW, DEGW)])


@functools.cache
def _make_sc(compute_deg):
    mesh = plsc.VectorSubcoreMesh(core_axis_name="c", subcore_axis_name="s")
    out_type = [jax.ShapeDtypeStruct((NC, NPAD, D), jnp.float32)]
    scratch = [
        pltpu.VMEM_SHARED((NPAD, D), jnp.float32),
    ]
    if compute_deg:
        out_type.append(jax.ShapeDtypeStruct((NC, NPAD), jnp.float32))
        scratch.append(pltpu.VMEM_SHARED((NS, NPAD), jnp.float32))
    scratch += [
        pltpu.VMEM((K,), jnp.int32),
        pltpu.VMEM((K,), jnp.int32),
        pltpu.VMEM((K,), jnp.float32),
        pltpu.VMEM((K, D), jnp.float32),
        pltpu.VMEM((32, D), jnp.float32),
    ]
    if compute_deg:
        scratch += [
            pltpu.VMEM((NPAD,), jnp.float32),
            pltpu.VMEM((NS, DEGW), jnp.float32),
            pltpu.VMEM((DEGW,), jnp.float32),
        ]
    scratch += [pltpu.SemaphoreType.DMA, pltpu.SemaphoreType.DMA]
    return pl.kernel(
        functools.partial(_sc_body, compute_deg),
        out_type=out_type,
        mesh=mesh,
        scratch_types=scratch,
        compiler_params=pltpu.CompilerParams(needs_layout_passes=False),
    )


def _tc_lin_body(y_ref, w_ref, b_ref, o_ref):
    o_ref[...] = jnp.dot(y_ref[...], w_ref[...],
                         preferred_element_type=jnp.float32) + b_ref[...]


def _tc_lin(y, W1, b1):
    B = 2000
    return pl.pallas_call(
        _tc_lin_body,
        grid=(N // B,),
        in_specs=[pl.BlockSpec((B, D), lambda i: (i, 0)),
                  pl.BlockSpec((D, D), lambda i: (0, 0)),
                  pl.BlockSpec((1, D), lambda i: (0, 0))],
        out_specs=pl.BlockSpec((B, D), lambda i: (i, 0)),
        out_shape=jax.ShapeDtypeStruct((N, D), jnp.float32),
    )(y, W1, b1.reshape(1, D))


def _combine(z_ref, deg_ref, y_ref, w2_ref, w3_ref, b3_ref):
    yv = y_ref[...]
    z = z_ref[0] + z_ref[1]
    deg = deg_ref[0] + deg_ref[1]
    t = (z - deg * jnp.dot(yv, w2_ref[...], preferred_element_type=jnp.float32)
         + jnp.dot(yv, w3_ref[...], preferred_element_type=jnp.float32)
         + b3_ref[...])
    return jnp.where(t >= 0, t, 0.01 * t)


def _tc_mid_body(z_ref, deg_ref, y_ref, w2_ref, w3_ref, b3_ref, w1n_ref,
                 b1n_ref, y1_ref, a1_ref):
    y1 = _combine(z_ref, deg_ref, y_ref, w2_ref, w3_ref, b3_ref)
    y1_ref[...] = y1
    a1_ref[...] = jnp.dot(y1, w1n_ref[...],
                          preferred_element_type=jnp.float32) + b1n_ref[...]


def _tc_mid(z, deg, y, W2, W3, b3, W1n, b1n):
    B = 2000
    return pl.pallas_call(
        _tc_mid_body,
        grid=(N // B,),
        in_specs=[pl.BlockSpec((NC, B, D), lambda i: (0, i, 0)),
                  pl.BlockSpec((NC, B, 1), lambda i: (0, i, 0)),
                  pl.BlockSpec((B, D), lambda i: (i, 0)),
                  pl.BlockSpec((D, D), lambda i: (0, 0)),
                  pl.BlockSpec((D, D), lambda i: (0, 0)),
                  pl.BlockSpec((1, D), lambda i: (0, 0)),
                  pl.BlockSpec((D, D), lambda i: (0, 0)),
                  pl.BlockSpec((1, D), lambda i: (0, 0))],
        out_specs=[pl.BlockSpec((B, D), lambda i: (i, 0)),
                   pl.BlockSpec((B, D), lambda i: (i, 0))],
        out_shape=[jax.ShapeDtypeStruct((N, D), jnp.float32),
                   jax.ShapeDtypeStruct((N, D), jnp.float32)],
    )(z, deg, y, W2, W3, b3.reshape(1, D), W1n, b1n.reshape(1, D))


def _tc_final_body(z_ref, deg_ref, y_ref, w2_ref, w3_ref, b3_ref, o_ref):
    o_ref[...] = _combine(z_ref, deg_ref, y_ref, w2_ref, w3_ref, b3_ref)


def _tc_final(z, deg, y, W2, W3, b3):
    B = 2000
    return pl.pallas_call(
        _tc_final_body,
        grid=(N // B,),
        in_specs=[pl.BlockSpec((NC, B, D), lambda i: (0, i, 0)),
                  pl.BlockSpec((NC, B, 1), lambda i: (0, i, 0)),
                  pl.BlockSpec((B, D), lambda i: (i, 0)),
                  pl.BlockSpec((D, D), lambda i: (0, 0)),
                  pl.BlockSpec((D, D), lambda i: (0, 0)),
                  pl.BlockSpec((1, D), lambda i: (0, 0))],
        out_specs=pl.BlockSpec((B, D), lambda i: (i, 0)),
        out_shape=jax.ShapeDtypeStruct((N, D), jnp.float32),
    )(z, deg, y, W2, W3, b3.reshape(1, D))


def kernel(y, edge_index, edge_weight,
           W1_0, b1_0, W2_0, W3_0, b3_0,
           W1_1, b1_1, W2_1, W3_1, b3_1):
    src = edge_index[0]
    dst = edge_index[1]
    a0 = _tc_lin(y, W1_0, b1_0)
    z0, degp = _make_sc(True)(a0, src, dst, edge_weight)
    deg = degp[:, :, None]
    y1, a1 = _tc_mid(z0, deg, y, W2_0, W3_0, b3_0, W1_1, b1_1)
    (z1,) = _make_sc(False)(a1, src, dst, edge_weight)
    return _tc_final(z1, deg, y1, W2_1, W3_1, b3_1)


# speedup vs baseline: 8.8661x; 1.6370x over previous
import functools

import jax
import jax.numpy as jnp
from jax import lax
from jax.experimental import pallas as pl
from jax.experimental.pallas import tpu as pltpu
from jax.experimental.pallas import tpu_sc as plsc

N = 10000
E = 320000
D = 128
NC = 2
NS = 16
NW = NC * NS
EP = E // NW
K = 80
NCHUNK = EP // K
NPAD = 10240
ZROWS = NPAD // NS
DEGW = NPAD // NS
LG = D // 16

_GATHER_DN = lax.GatherDimensionNumbers(
    offset_dims=(), collapsed_slice_dims=(0,), start_index_map=(0,))


def _splat(v16, e):
    return lax.gather(v16, jnp.full((16, 1), e, jnp.int32), _GATHER_DN, (1,),
                      mode=lax.GatherScatterMode.PROMISE_IN_BOUNDS)


def _sc_body(compute_deg, a_hbm, idx_hbm, w_hbm, *rest):
    if compute_deg:
        (z_out, deg_out, zsh, dsh, ib0, ib1, ib2, wb0, wb1, wb2,
         rows0, rows1, wstage, zbuf, zd,
         is0, is1, is2, g0, g1, s0, s1, d0, d1, d2) = rest
        dsem = (d0, d1, d2)
    else:
        (z_out, zsh, ib0, ib1, ib2, wb0, wb1, wb2,
         rows0, rows1, wstage, zbuf,
         is0, is1, is2, g0, g1, s0, s1) = rest
    ib = (ib0, ib1, ib2)
    wb = (wb0, wb1, wb2)
    isem = (is0, is1, is2)
    rowsb = (rows0, rows1)
    gsem = (g0, g1)
    ssem = (s0, s1)
    c = lax.axis_index("c")
    s = lax.axis_index("s")
    wid = s * NC + c
    zero16 = jnp.zeros((16,), jnp.float32)

    pltpu.sync_copy(idx_hbm.at[wid, 0], ib0)
    pltpu.sync_copy(w_hbm.at[wid, 0], wb0)
    pltpu.sync_copy(idx_hbm.at[wid, 1], ib1)
    pltpu.sync_copy(w_hbm.at[wid, 1], wb1)
    pltpu.async_copy(a_hbm.at[ib0.at[0]], rows0, g0)

    def zero_zbuf(i, _):
        for j in range(LG):
            zbuf[i, pl.ds(j * 16, 16)] = zero16
        return 0
    lax.fori_loop(0, 8, zero_zbuf, 0)

    def zero_zsh(k, _):
        pltpu.sync_copy(zbuf, zsh.at[pl.ds(s * ZROWS + k * 8, 8)])
        return 0
    lax.fori_loop(0, ZROWS // 8, zero_zsh, 0)

    if compute_deg:
        def zero_zd(i, _):
            zd[pl.ds(i * 16, 16)] = zero16
            return 0
        lax.fori_loop(0, DEGW // 16, zero_zd, 0)
        pltpu.sync_copy(zd, dsh.at[pl.ds(s * DEGW, DEGW)])

    plsc.subcore_barrier()

    def scale(b):
        rows = rowsb[b]
        for g in range(K // 16):
            w16 = wstage[pl.ds(g * 16, 16)]
            for e in range(16):
                r = g * 16 + e
                we = _splat(w16, e)
                for j in range(LG):
                    rows[r, pl.ds(j * 16, 16)] = rows[r, pl.ds(j * 16, 16)] * we

    def chunk(i, _):
        for r in range(6):
            @pl.when(i % 6 == r)
            def _(r=r):
                b = r % 2
                t = r % 3
                nb = 1 - b
                t1 = (r + 1) % 3
                t2 = (r + 2) % 3

                @pl.when(i >= 1)
                def _():
                    pltpu.make_async_copy(rowsb[nb], zsh.at[ib0.at[1]],
                                          ssem[nb]).wait()
                    if compute_deg:
                        pltpu.make_async_copy(wb0, dsh.at[ib0.at[1]],
                                              dsem[t2]).wait()

                @pl.when(i + 2 < NCHUNK)
                def _():
                    pltpu.async_copy(idx_hbm.at[wid, i + 2], ib[t2], isem[t2])
                    pltpu.async_copy(w_hbm.at[wid, i + 2], wb[t2], isem[t2])

                @pl.when(jnp.logical_and(i + 1 < NCHUNK, i >= 1))
                def _():
                    pltpu.make_async_copy(idx_hbm.at[wid, 0], ib[t1],
                                          isem[t1]).wait()
                    pltpu.make_async_copy(w_hbm.at[wid, 0], wb[t1],
                                          isem[t1]).wait()

                @pl.when(i + 1 < NCHUNK)
                def _():
                    pltpu.async_copy(a_hbm.at[ib[t1].at[0]], rowsb[nb],
                                     gsem[nb])

                pltpu.make_async_copy(a_hbm.at[ib0.at[0]], rowsb[b],
                                      gsem[b]).wait()
                for g in range(K // 16):
                    wstage[pl.ds(g * 16, 16)] = wb[t][pl.ds(g * 16, 16)]
                scale(b)
                pltpu.async_copy(rowsb[b], zsh.at[ib[t].at[1]], ssem[b],
                                 add=True)
                if compute_deg:
                    pltpu.async_copy(wb[t], dsh.at[ib[t].at[1]], dsem[t],
                                     add=True)
        return 0
    lax.fori_loop(0, NCHUNK, chunk, 0)

    lb = (NCHUNK - 1) % 2
    lt = (NCHUNK - 1) % 3
    pltpu.make_async_copy(rowsb[lb], zsh.at[ib0.at[1]], ssem[lb]).wait()
    if compute_deg:
        pltpu.make_async_copy(wb0, dsh.at[ib0.at[1]], dsem[lt]).wait()

    plsc.subcore_barrier()

    pltpu.sync_copy(zsh.at[pl.ds(s * ZROWS, ZROWS)],
                    z_out.at[c, pl.ds(s * ZROWS, ZROWS)])
    if compute_deg:
        pltpu.sync_copy(dsh.at[pl.ds(s * DEGW, DEGW)],
                        deg_out.at[c, pl.ds(s * DEGW, DEGW)])


@functools.cache
def _make_sc(compute_deg):
    mesh = plsc.VectorSubcoreMesh(core_axis_name="c", subcore_axis_name="s")
    out_type = [jax.ShapeDtypeStruct((NC, NPAD, D), jnp.float32)]
    scratch = [pltpu.VMEM_SHARED((NPAD, D), jnp.float32)]
    if compute_deg:
        out_type.append(jax.ShapeDtypeStruct((NC, NPAD), jnp.float32))
        scratch.append(pltpu.VMEM_SHARED((NPAD,), jnp.float32))
    scratch += [
        pltpu.VMEM((2, K), jnp.int32),
        pltpu.VMEM((2, K), jnp.int32),
        pltpu.VMEM((2, K), jnp.int32),
        pltpu.VMEM((K,), jnp.float32),
        pltpu.VMEM((K,), jnp.float32),
        pltpu.VMEM((K,), jnp.float32),
        pltpu.VMEM((K, D), jnp.float32),
        pltpu.VMEM((K, D), jnp.float32),
        pltpu.VMEM((K,), jnp.float32),
        pltpu.VMEM((8, D), jnp.float32),
    ]
    if compute_deg:
        scratch.append(pltpu.VMEM((DEGW,), jnp.float32))
    nsem = 10 if compute_deg else 7
    scratch += [pltpu.SemaphoreType.DMA] * nsem
    return pl.kernel(
        functools.partial(_sc_body, compute_deg),
        out_type=out_type,
        mesh=mesh,
        scratch_types=scratch,
        compiler_params=pltpu.CompilerParams(needs_layout_passes=False),
    )


def _tc_lin_body(y_ref, w_ref, b_ref, o_ref):
    o_ref[...] = jnp.dot(y_ref[...], w_ref[...],
                         preferred_element_type=jnp.float32) + b_ref[...]


def _tc_lin(y, W1, b1):
    B = 2000
    return pl.pallas_call(
        _tc_lin_body,
        grid=(N // B,),
        in_specs=[pl.BlockSpec((B, D), lambda i: (i, 0)),
                  pl.BlockSpec((D, D), lambda i: (0, 0)),
                  pl.BlockSpec((1, D), lambda i: (0, 0))],
        out_specs=pl.BlockSpec((B, D), lambda i: (i, 0)),
        out_shape=jax.ShapeDtypeStruct((N, D), jnp.float32),
    )(y, W1, b1.reshape(1, D))


def _combine(z_ref, deg_ref, y_ref, w2_ref, w3_ref, b3_ref):
    yv = y_ref[...]
    z = z_ref[0] + z_ref[1]
    deg = deg_ref[0] + deg_ref[1]
    t = (z - deg * jnp.dot(yv, w2_ref[...], preferred_element_type=jnp.float32)
         + jnp.dot(yv, w3_ref[...], preferred_element_type=jnp.float32)
         + b3_ref[...])
    return jnp.where(t >= 0, t, 0.01 * t)


def _tc_mid_body(z_ref, deg_ref, y_ref, w2_ref, w3_ref, b3_ref, w1n_ref,
                 b1n_ref, y1_ref, a1_ref):
    y1 = _combine(z_ref, deg_ref, y_ref, w2_ref, w3_ref, b3_ref)
    y1_ref[...] = y1
    a1_ref[...] = jnp.dot(y1, w1n_ref[...],
                          preferred_element_type=jnp.float32) + b1n_ref[...]


def _tc_mid(z, deg, y, W2, W3, b3, W1n, b1n):
    B = 2000
    return pl.pallas_call(
        _tc_mid_body,
        grid=(N // B,),
        in_specs=[pl.BlockSpec((NC, B, D), lambda i: (0, i, 0)),
                  pl.BlockSpec((NC, B, 1), lambda i: (0, i, 0)),
                  pl.BlockSpec((B, D), lambda i: (i, 0)),
                  pl.BlockSpec((D, D), lambda i: (0, 0)),
                  pl.BlockSpec((D, D), lambda i: (0, 0)),
                  pl.BlockSpec((1, D), lambda i: (0, 0)),
                  pl.BlockSpec((D, D), lambda i: (0, 0)),
                  pl.BlockSpec((1, D), lambda i: (0, 0))],
        out_specs=[pl.BlockSpec((B, D), lambda i: (i, 0)),
                   pl.BlockSpec((B, D), lambda i: (i, 0))],
        out_shape=[jax.ShapeDtypeStruct((N, D), jnp.float32),
                   jax.ShapeDtypeStruct((N, D), jnp.float32)],
    )(z, deg, y, W2, W3, b3.reshape(1, D), W1n, b1n.reshape(1, D))


def _tc_final_body(z_ref, deg_ref, y_ref, w2_ref, w3_ref, b3_ref, o_ref):
    o_ref[...] = _combine(z_ref, deg_ref, y_ref, w2_ref, w3_ref, b3_ref)


def _tc_final(z, deg, y, W2, W3, b3):
    B = 2000
    return pl.pallas_call(
        _tc_final_body,
        grid=(N // B,),
        in_specs=[pl.BlockSpec((NC, B, D), lambda i: (0, i, 0)),
                  pl.BlockSpec((NC, B, 1), lambda i: (0, i, 0)),
                  pl.BlockSpec((B, D), lambda i: (i, 0)),
                  pl.BlockSpec((D, D), lambda i: (0, 0)),
                  pl.BlockSpec((D, D), lambda i: (0, 0)),
                  pl.BlockSpec((1, D), lambda i: (0, 0))],
        out_specs=pl.BlockSpec((B, D), lambda i: (i, 0)),
        out_shape=jax.ShapeDtypeStruct((N, D), jnp.float32),
    )(z, deg, y, W2, W3, b3.reshape(1, D))


def kernel(y, edge_index, edge_weight,
           W1_0, b1_0, W2_0, W3_0, b3_0,
           W1_1, b1_1, W2_1, W3_1, b3_1):
    idx_p = edge_index.reshape(2, NW, NCHUNK, K).transpose(1, 2, 0, 3)
    w_p = edge_weight.reshape(NW, NCHUNK, K)
    a0 = _tc_lin(y, W1_0, b1_0)
    z0, degp = _make_sc(True)(a0, idx_p, w_p)
    deg = degp[:, :, None]
    y1, a1 = _tc_mid(z0, deg, y, W2_0, W3_0, b3_0, W1_1, b1_1)
    (z1,) = _make_sc(False)(a1, idx_p, w_p)
    return _tc_final(z1, deg, y1, W2_1, W3_1, b3_1)


# speedup vs baseline: 10.0030x; 1.1282x over previous
import functools

import jax
import jax.numpy as jnp
from jax import lax
from jax.experimental import pallas as pl
from jax.experimental.pallas import tpu as pltpu
from jax.experimental.pallas import tpu_sc as plsc

N = 10000
E = 320000
D = 128
NC = 2
NS = 16
NW = NC * NS
EP = E // NW
K = 80
NCHUNK = EP // K
NPAD = 10240
ZROWS = NPAD // NS
DEGW = NPAD // NS
LG = D // 16

_GATHER_DN = lax.GatherDimensionNumbers(
    offset_dims=(), collapsed_slice_dims=(0,), start_index_map=(0,))


def _splat(v16, e):
    return lax.gather(v16, jnp.full((16, 1), e, jnp.int32), _GATHER_DN, (1,),
                      mode=lax.GatherScatterMode.PROMISE_IN_BOUNDS)


def _sc_body(compute_deg, a_hbm, idx_hbm, *rest):
    if compute_deg:
        (z_out, deg_out, zsh, dsh, ib0, ib1, ib2, ib3,
         rows0, rows1, rows2, ws0, ws1, ws2, ds0, ds1, ds2, zbuf, zd,
         i0, i1, i2, i3, g0, g1, g2, s0, s1, s2, d0, d1, d2) = rest
        dsem = (d0, d1, d2)
    else:
        (z_out, zsh, ib0, ib1, ib2, ib3,
         rows0, rows1, rows2, ws0, ws1, ws2, ds0, ds1, ds2, zbuf,
         i0, i1, i2, i3, g0, g1, g2, s0, s1, s2) = rest
    ib = (ib0, ib1, ib2, ib3)
    isem = (i0, i1, i2, i3)
    rowsb = (rows0, rows1, rows2)
    wstage = (ws0, ws1, ws2)
    dstage = (ds0, ds1, ds2)
    gsem = (g0, g1, g2)
    ssem = (s0, s1, s2)
    c = lax.axis_index("c")
    s = lax.axis_index("s")
    wid = s * NC + c
    zero16 = jnp.zeros((16,), jnp.float32)

    pltpu.sync_copy(idx_hbm.at[wid, 0], ib0)
    pltpu.sync_copy(idx_hbm.at[wid, 1], ib1)
    pltpu.async_copy(a_hbm.at[ib0.at[0]], rows0, g0)

    def zero_zbuf(i, _):
        for j in range(LG):
            zbuf[i, pl.ds(j * 16, 16)] = zero16
        return 0
    lax.fori_loop(0, 8, zero_zbuf, 0)

    def zero_zsh(k, _):
        pltpu.sync_copy(zbuf, zsh.at[pl.ds(s * ZROWS + k * 8, 8)])
        return 0
    lax.fori_loop(0, ZROWS // 8, zero_zsh, 0)

    if compute_deg:
        def zero_zd(i, _):
            zd[pl.ds(i * 16, 16)] = zero16
            return 0
        lax.fori_loop(0, DEGW // 16, zero_zd, 0)
        pltpu.sync_copy(zd, dsh.at[pl.ds(s * DEGW, DEGW)])

    plsc.subcore_barrier()

    def scale(t):
        rows = rowsb[t]
        ws = wstage[t]
        for g in range(K // 16):
            w16 = ws[pl.ds(g * 16, 16)]
            for e in range(16):
                r = g * 16 + e
                we = _splat(w16, e)
                for j in range(LG):
                    rows[r, pl.ds(j * 16, 16)] = rows[r, pl.ds(j * 16, 16)] * we

    def chunk(i, _):
        for t3 in range(3):
            @pl.when(jnp.logical_and(i % 3 == t3, i >= 2))
            def _(t3=t3):
                tn = (t3 + 1) % 3
                pltpu.make_async_copy(rowsb[tn], zsh.at[ib0.at[1]],
                                      ssem[tn]).wait()
                if compute_deg:
                    pltpu.make_async_copy(ws0, dsh.at[ib0.at[1]],
                                          dsem[tn]).wait()

        for r in range(12):
            @pl.when(i % 12 == r)
            def _(r=r):
                t = r % 3
                q = r % 4
                q2 = (r + 2) % 4
                q1 = (r + 1) % 4
                tnext = (r + 1) % 3

                @pl.when(i + 2 < NCHUNK)
                def _():
                    pltpu.async_copy(idx_hbm.at[wid, i + 2], ib[q2], isem[q2])

                @pl.when(jnp.logical_and(i + 1 < NCHUNK, i >= 1))
                def _():
                    pltpu.make_async_copy(idx_hbm.at[wid, 0], ib[q1],
                                          isem[q1]).wait()

                @pl.when(i + 1 < NCHUNK)
                def _():
                    pltpu.async_copy(a_hbm.at[ib[q1].at[0]], rowsb[tnext],
                                     gsem[tnext])

                for g in range(K // 16):
                    wstage[t][pl.ds(g * 16, 16)] = plsc.bitcast(
                        ib[q][2, pl.ds(g * 16, 16)], jnp.float32)
                    dstage[t][pl.ds(g * 16, 16)] = ib[q][1, pl.ds(g * 16, 16)]

        for t3 in range(3):
            @pl.when(i % 3 == t3)
            def _(t3=t3):
                pltpu.make_async_copy(a_hbm.at[ib0.at[0]], rowsb[t3],
                                      gsem[t3]).wait()
                scale(t3)
                pltpu.async_copy(rowsb[t3], zsh.at[dstage[t3]], ssem[t3],
                                 add=True)
                if compute_deg:
                    pltpu.async_copy(wstage[t3], dsh.at[dstage[t3]], dsem[t3],
                                     add=True)
        return 0
    lax.fori_loop(0, NCHUNK, chunk, 0)

    for lc in (NCHUNK - 2, NCHUNK - 1):
        lt = lc % 3
        pltpu.make_async_copy(rowsb[lt], zsh.at[ib0.at[1]], ssem[lt]).wait()
        if compute_deg:
            pltpu.make_async_copy(ws0, dsh.at[ib0.at[1]], dsem[lt]).wait()

    plsc.subcore_barrier()

    pltpu.sync_copy(zsh.at[pl.ds(s * ZROWS, ZROWS)],
                    z_out.at[c, pl.ds(s * ZROWS, ZROWS)])
    if compute_deg:
        pltpu.sync_copy(dsh.at[pl.ds(s * DEGW, DEGW)],
                        deg_out.at[c, pl.ds(s * DEGW, DEGW)])


@functools.cache
def _make_sc(compute_deg):
    mesh = plsc.VectorSubcoreMesh(core_axis_name="c", subcore_axis_name="s")
    out_type = [jax.ShapeDtypeStruct((NC, NPAD, D), jnp.float32)]
    scratch = [pltpu.VMEM_SHARED((NPAD, D), jnp.float32)]
    if compute_deg:
        out_type.append(jax.ShapeDtypeStruct((NC, NPAD), jnp.float32))
        scratch.append(pltpu.VMEM_SHARED((NPAD,), jnp.float32))
    scratch += [
        pltpu.VMEM((3, K), jnp.int32),
        pltpu.VMEM((3, K), jnp.int32),
        pltpu.VMEM((3, K), jnp.int32),
        pltpu.VMEM((3, K), jnp.int32),
        pltpu.VMEM((K, D), jnp.float32),
        pltpu.VMEM((K, D), jnp.float32),
        pltpu.VMEM((K, D), jnp.float32),
        pltpu.VMEM((K,), jnp.float32),
        pltpu.VMEM((K,), jnp.float32),
        pltpu.VMEM((K,), jnp.float32),
        pltpu.VMEM((K,), jnp.int32),
        pltpu.VMEM((K,), jnp.int32),
        pltpu.VMEM((K,), jnp.int32),
        pltpu.VMEM((8, D), jnp.float32),
    ]
    if compute_deg:
        scratch.append(pltpu.VMEM((DEGW,), jnp.float32))
    nsem = 13 if compute_deg else 10
    scratch += [pltpu.SemaphoreType.DMA] * nsem
    return pl.kernel(
        functools.partial(_sc_body, compute_deg),
        out_type=out_type,
        mesh=mesh,
        scratch_types=scratch,
        compiler_params=pltpu.CompilerParams(needs_layout_passes=False),
    )


def _tc_lin_body(y_ref, w_ref, b_ref, o_ref):
    o_ref[...] = jnp.dot(y_ref[...], w_ref[...],
                         preferred_element_type=jnp.float32) + b_ref[...]


def _tc_lin(y, W1, b1):
    B = 2000
    return pl.pallas_call(
        _tc_lin_body,
        grid=(N // B,),
        in_specs=[pl.BlockSpec((B, D), lambda i: (i, 0)),
                  pl.BlockSpec((D, D), lambda i: (0, 0)),
                  pl.BlockSpec((1, D), lambda i: (0, 0))],
        out_specs=pl.BlockSpec((B, D), lambda i: (i, 0)),
        out_shape=jax.ShapeDtypeStruct((N, D), jnp.float32),
    )(y, W1, b1.reshape(1, D))


def _combine(z_ref, deg_ref, y_ref, w2_ref, w3_ref, b3_ref):
    yv = y_ref[...]
    z = z_ref[0] + z_ref[1]
    deg = deg_ref[0] + deg_ref[1]
    t = (z - deg * jnp.dot(yv, w2_ref[...], preferred_element_type=jnp.float32)
         + jnp.dot(yv, w3_ref[...], preferred_element_type=jnp.float32)
         + b3_ref[...])
    return jnp.where(t >= 0, t, 0.01 * t)


def _tc_mid_body(z_ref, deg_ref, y_ref, w2_ref, w3_ref, b3_ref, w1n_ref,
                 b1n_ref, y1_ref, a1_ref):
    y1 = _combine(z_ref, deg_ref, y_ref, w2_ref, w3_ref, b3_ref)
    y1_ref[...] = y1
    a1_ref[...] = jnp.dot(y1, w1n_ref[...],
                          preferred_element_type=jnp.float32) + b1n_ref[...]


def _tc_mid(z, deg, y, W2, W3, b3, W1n, b1n):
    B = 2000
    return pl.pallas_call(
        _tc_mid_body,
        grid=(N // B,),
        in_specs=[pl.BlockSpec((NC, B, D), lambda i: (0, i, 0)),
                  pl.BlockSpec((NC, B, 1), lambda i: (0, i, 0)),
                  pl.BlockSpec((B, D), lambda i: (i, 0)),
                  pl.BlockSpec((D, D), lambda i: (0, 0)),
                  pl.BlockSpec((D, D), lambda i: (0, 0)),
                  pl.BlockSpec((1, D), lambda i: (0, 0)),
                  pl.BlockSpec((D, D), lambda i: (0, 0)),
                  pl.BlockSpec((1, D), lambda i: (0, 0))],
        out_specs=[pl.BlockSpec((B, D), lambda i: (i, 0)),
                   pl.BlockSpec((B, D), lambda i: (i, 0))],
        out_shape=[jax.ShapeDtypeStruct((N, D), jnp.float32),
                   jax.ShapeDtypeStruct((N, D), jnp.float32)],
    )(z, deg, y, W2, W3, b3.reshape(1, D), W1n, b1n.reshape(1, D))


def _tc_final_body(z_ref, deg_ref, y_ref, w2_ref, w3_ref, b3_ref, o_ref):
    o_ref[...] = _combine(z_ref, deg_ref, y_ref, w2_ref, w3_ref, b3_ref)


def _tc_final(z, deg, y, W2, W3, b3):
    B = 2000
    return pl.pallas_call(
        _tc_final_body,
        grid=(N // B,),
        in_specs=[pl.BlockSpec((NC, B, D), lambda i: (0, i, 0)),
                  pl.BlockSpec((NC, B, 1), lambda i: (0, i, 0)),
                  pl.BlockSpec((B, D), lambda i: (i, 0)),
                  pl.BlockSpec((D, D), lambda i: (0, 0)),
                  pl.BlockSpec((D, D), lambda i: (0, 0)),
                  pl.BlockSpec((1, D), lambda i: (0, 0))],
        out_specs=pl.BlockSpec((B, D), lambda i: (i, 0)),
        out_shape=jax.ShapeDtypeStruct((N, D), jnp.float32),
    )(z, deg, y, W2, W3, b3.reshape(1, D))


def kernel(y, edge_index, edge_weight,
           W1_0, b1_0, W2_0, W3_0, b3_0,
           W1_1, b1_1, W2_1, W3_1, b3_1):
    w_bits = lax.bitcast_convert_type(edge_weight, jnp.int32)
    idx_p = jnp.stack(
        [edge_index[0].reshape(NW, NCHUNK, K),
         edge_index[1].reshape(NW, NCHUNK, K),
         w_bits.reshape(NW, NCHUNK, K)], axis=2)
    a0 = _tc_lin(y, W1_0, b1_0)
    z0, degp = _make_sc(True)(a0, idx_p)
    deg = degp[:, :, None]
    y1, a1 = _tc_mid(z0, deg, y, W2_0, W3_0, b3_0, W1_1, b1_1)
    (z1,) = _make_sc(False)(a1, idx_p)
    return _tc_final(z1, deg, y1, W2_1, W3_1, b3_1)


# speedup vs baseline: 10.1334x; 1.0130x over previous
import functools

import jax
import jax.numpy as jnp
from jax import lax
from jax.experimental import pallas as pl
from jax.experimental.pallas import tpu as pltpu
from jax.experimental.pallas import tpu_sc as plsc

N = 10000
E = 320000
D = 128
NC = 2
NS = 16
NW = NC * NS
EP = E // NW
K = 80
NCHUNK = EP // K
NPAD = 10240
ZROWS = NPAD // NS
DEGW = NPAD // NS
LG = D // 16

_GATHER_DN = lax.GatherDimensionNumbers(
    offset_dims=(), collapsed_slice_dims=(0,), start_index_map=(0,))


def _splat(v16, e):
    return lax.gather(v16, jnp.full((16, 1), e, jnp.int32), _GATHER_DN, (1,),
                      mode=lax.GatherScatterMode.PROMISE_IN_BOUNDS)


def _sc_body(compute_deg, a_hbm, idx_hbm, *rest):
    if compute_deg:
        (z_out, deg_out, zsh, dsh, ib0, ib1, ib2, ib3,
         rows0, rows1, rows2, ws0, ws1, ws2, ds0, ds1, ds2, zbuf, zd,
         i0, i1, i2, i3, g0, g1, g2, s0, s1, s2, d0, d1, d2, zsem) = rest
        dsem = (d0, d1, d2)
    else:
        (z_out, zsh, ib0, ib1, ib2, ib3,
         rows0, rows1, rows2, ws0, ws1, ws2, ds0, ds1, ds2, zbuf,
         i0, i1, i2, i3, g0, g1, g2, s0, s1, s2, zsem) = rest
    ib = (ib0, ib1, ib2, ib3)
    isem = (i0, i1, i2, i3)
    rowsb = (rows0, rows1, rows2)
    wstage = (ws0, ws1, ws2)
    dstage = (ds0, ds1, ds2)
    gsem = (g0, g1, g2)
    ssem = (s0, s1, s2)
    c = lax.axis_index("c")
    s = lax.axis_index("s")
    wid = s * NC + c
    zero16 = jnp.zeros((16,), jnp.float32)

    pltpu.sync_copy(idx_hbm.at[wid, 0], ib0)
    pltpu.sync_copy(idx_hbm.at[wid, 1], ib1)
    pltpu.async_copy(a_hbm.at[ib0.at[0]], rows0, g0)

    def zero_zbuf(i, _):
        for j in range(LG):
            zbuf[i, pl.ds(j * 16, 16)] = zero16
        return 0
    lax.fori_loop(0, 64, zero_zbuf, 0)

    def zero_zsh(k, _):
        pltpu.async_copy(zbuf, zsh.at[pl.ds(s * ZROWS + k * 64, 64)], zsem)
        return 0
    lax.fori_loop(0, ZROWS // 64, zero_zsh, 0)

    if compute_deg:
        def zero_zd(i, _):
            zd[pl.ds(i * 16, 16)] = zero16
            return 0
        lax.fori_loop(0, DEGW // 16, zero_zd, 0)
        pltpu.async_copy(zd, dsh.at[pl.ds(s * DEGW, DEGW)], zsem)

    def drain_zero(k, _):
        pltpu.make_async_copy(zbuf, zsh.at[pl.ds(s * ZROWS, 64)], zsem).wait()
        return 0
    lax.fori_loop(0, ZROWS // 64, drain_zero, 0)
    if compute_deg:
        pltpu.make_async_copy(zd, dsh.at[pl.ds(s * DEGW, DEGW)], zsem).wait()

    plsc.subcore_barrier()

    def scale(t):
        rows = rowsb[t]
        ws = wstage[t]
        for g in range(K // 16):
            w16 = ws[pl.ds(g * 16, 16)]
            for e in range(16):
                r = g * 16 + e
                we = _splat(w16, e)
                for j in range(LG):
                    rows[r, pl.ds(j * 16, 16)] = rows[r, pl.ds(j * 16, 16)] * we

    def chunk(i, _):
        for t3 in range(3):
            @pl.when(jnp.logical_and(i % 3 == t3, i >= 2))
            def _(t3=t3):
                tn = (t3 + 1) % 3
                pltpu.make_async_copy(rowsb[tn], zsh.at[ib0.at[1]],
                                      ssem[tn]).wait()
                if compute_deg:
                    pltpu.make_async_copy(ws0, dsh.at[ib0.at[1]],
                                          dsem[tn]).wait()

        for r in range(12):
            @pl.when(i % 12 == r)
            def _(r=r):
                t = r % 3
                q = r % 4
                q2 = (r + 2) % 4
                q1 = (r + 1) % 4
                tnext = (r + 1) % 3

                @pl.when(i + 2 < NCHUNK)
                def _():
                    pltpu.async_copy(idx_hbm.at[wid, i + 2], ib[q2], isem[q2])

                @pl.when(jnp.logical_and(i + 1 < NCHUNK, i >= 1))
                def _():
                    pltpu.make_async_copy(idx_hbm.at[wid, 0], ib[q1],
                                          isem[q1]).wait()

                @pl.when(i + 1 < NCHUNK)
                def _():
                    pltpu.async_copy(a_hbm.at[ib[q1].at[0]], rowsb[tnext],
                                     gsem[tnext])

                for g in range(K // 16):
                    wstage[t][pl.ds(g * 16, 16)] = plsc.bitcast(
                        ib[q][2, pl.ds(g * 16, 16)], jnp.float32)
                    dstage[t][pl.ds(g * 16, 16)] = ib[q][1, pl.ds(g * 16, 16)]

        for t3 in range(3):
            @pl.when(i % 3 == t3)
            def _(t3=t3):
                pltpu.make_async_copy(a_hbm.at[ib0.at[0]], rowsb[t3],
                                      gsem[t3]).wait()
                scale(t3)
                pltpu.async_copy(rowsb[t3], zsh.at[dstage[t3]], ssem[t3],
                                 add=True)
                if compute_deg:
                    pltpu.async_copy(wstage[t3], dsh.at[dstage[t3]], dsem[t3],
                                     add=True)
        return 0
    lax.fori_loop(0, NCHUNK, chunk, 0)

    for lc in (NCHUNK - 2, NCHUNK - 1):
        lt = lc % 3
        pltpu.make_async_copy(rowsb[lt], zsh.at[ib0.at[1]], ssem[lt]).wait()
        if compute_deg:
            pltpu.make_async_copy(ws0, dsh.at[ib0.at[1]], dsem[lt]).wait()

    plsc.subcore_barrier()

    pltpu.sync_copy(zsh.at[pl.ds(s * ZROWS, ZROWS)],
                    z_out.at[c, pl.ds(s * ZROWS, ZROWS)])
    if compute_deg:
        pltpu.sync_copy(dsh.at[pl.ds(s * DEGW, DEGW)],
                        deg_out.at[c, pl.ds(s * DEGW, DEGW)])


@functools.cache
def _make_sc(compute_deg):
    mesh = plsc.VectorSubcoreMesh(core_axis_name="c", subcore_axis_name="s")
    out_type = [jax.ShapeDtypeStruct((NC, NPAD, D), jnp.float32)]
    scratch = [pltpu.VMEM_SHARED((NPAD, D), jnp.float32)]
    if compute_deg:
        out_type.append(jax.ShapeDtypeStruct((NC, NPAD), jnp.float32))
        scratch.append(pltpu.VMEM_SHARED((NPAD,), jnp.float32))
    scratch += [
        pltpu.VMEM((3, K), jnp.int32),
        pltpu.VMEM((3, K), jnp.int32),
        pltpu.VMEM((3, K), jnp.int32),
        pltpu.VMEM((3, K), jnp.int32),
        pltpu.VMEM((K, D), jnp.float32),
        pltpu.VMEM((K, D), jnp.float32),
        pltpu.VMEM((K, D), jnp.float32),
        pltpu.VMEM((K,), jnp.float32),
        pltpu.VMEM((K,), jnp.float32),
        pltpu.VMEM((K,), jnp.float32),
        pltpu.VMEM((K,), jnp.int32),
        pltpu.VMEM((K,), jnp.int32),
        pltpu.VMEM((K,), jnp.int32),
        pltpu.VMEM((64, D), jnp.float32),
    ]
    if compute_deg:
        scratch.append(pltpu.VMEM((DEGW,), jnp.float32))
    nsem = 14 if compute_deg else 11
    scratch += [pltpu.SemaphoreType.DMA] * nsem
    return pl.kernel(
        functools.partial(_sc_body, compute_deg),
        out_type=out_type,
        mesh=mesh,
        scratch_types=scratch,
        compiler_params=pltpu.CompilerParams(needs_layout_passes=False),
    )


def _tc_lin_body(y_ref, w_ref, b_ref, o_ref):
    o_ref[...] = jnp.dot(y_ref[...], w_ref[...],
                         preferred_element_type=jnp.float32) + b_ref[...]


def _tc_lin(y, W1, b1):
    B = 2000
    return pl.pallas_call(
        _tc_lin_body,
        grid=(N // B,),
        in_specs=[pl.BlockSpec((B, D), lambda i: (i, 0)),
                  pl.BlockSpec((D, D), lambda i: (0, 0)),
                  pl.BlockSpec((1, D), lambda i: (0, 0))],
        out_specs=pl.BlockSpec((B, D), lambda i: (i, 0)),
        out_shape=jax.ShapeDtypeStruct((N, D), jnp.float32),
    )(y, W1, b1.reshape(1, D))


def _combine(z_ref, deg_ref, y_ref, w2_ref, w3_ref, b3_ref):
    yv = y_ref[...]
    z = z_ref[0] + z_ref[1]
    deg = deg_ref[0] + deg_ref[1]
    t = (z - deg * jnp.dot(yv, w2_ref[...], preferred_element_type=jnp.float32)
         + jnp.dot(yv, w3_ref[...], preferred_element_type=jnp.float32)
         + b3_ref[...])
    return jnp.where(t >= 0, t, 0.01 * t)


def _tc_mid_body(z_ref, deg_ref, y_ref, w2_ref, w3_ref, b3_ref, w1n_ref,
                 b1n_ref, y1_ref, a1_ref):
    y1 = _combine(z_ref, deg_ref, y_ref, w2_ref, w3_ref, b3_ref)
    y1_ref[...] = y1
    a1_ref[...] = jnp.dot(y1, w1n_ref[...],
                          preferred_element_type=jnp.float32) + b1n_ref[...]


def _tc_mid(z, deg, y, W2, W3, b3, W1n, b1n):
    B = 2000
    return pl.pallas_call(
        _tc_mid_body,
        grid=(N // B,),
        in_specs=[pl.BlockSpec((NC, B, D), lambda i: (0, i, 0)),
                  pl.BlockSpec((NC, B, 1), lambda i: (0, i, 0)),
                  pl.BlockSpec((B, D), lambda i: (i, 0)),
                  pl.BlockSpec((D, D), lambda i: (0, 0)),
                  pl.BlockSpec((D, D), lambda i: (0, 0)),
                  pl.BlockSpec((1, D), lambda i: (0, 0)),
                  pl.BlockSpec((D, D), lambda i: (0, 0)),
                  pl.BlockSpec((1, D), lambda i: (0, 0))],
        out_specs=[pl.BlockSpec((B, D), lambda i: (i, 0)),
                   pl.BlockSpec((B, D), lambda i: (i, 0))],
        out_shape=[jax.ShapeDtypeStruct((N, D), jnp.float32),
                   jax.ShapeDtypeStruct((N, D), jnp.float32)],
    )(z, deg, y, W2, W3, b3.reshape(1, D), W1n, b1n.reshape(1, D))


def _tc_final_body(z_ref, deg_ref, y_ref, w2_ref, w3_ref, b3_ref, o_ref):
    o_ref[...] = _combine(z_ref, deg_ref, y_ref, w2_ref, w3_ref, b3_ref)


def _tc_final(z, deg, y, W2, W3, b3):
    B = 2000
    return pl.pallas_call(
        _tc_final_body,
        grid=(N // B,),
        in_specs=[pl.BlockSpec((NC, B, D), lambda i: (0, i, 0)),
                  pl.BlockSpec((NC, B, 1), lambda i: (0, i, 0)),
                  pl.BlockSpec((B, D), lambda i: (i, 0)),
                  pl.BlockSpec((D, D), lambda i: (0, 0)),
                  pl.BlockSpec((D, D), lambda i: (0, 0)),
                  pl.BlockSpec((1, D), lambda i: (0, 0))],
        out_specs=pl.BlockSpec((B, D), lambda i: (i, 0)),
        out_shape=jax.ShapeDtypeStruct((N, D), jnp.float32),
    )(z, deg, y, W2, W3, b3.reshape(1, D))


def kernel(y, edge_index, edge_weight,
           W1_0, b1_0, W2_0, W3_0, b3_0,
           W1_1, b1_1, W2_1, W3_1, b3_1):
    w_bits = lax.bitcast_convert_type(edge_weight, jnp.int32)
    idx_p = jnp.stack(
        [edge_index[0].reshape(NW, NCHUNK, K),
         edge_index[1].reshape(NW, NCHUNK, K),
         w_bits.reshape(NW, NCHUNK, K)], axis=2)
    a0 = _tc_lin(y, W1_0, b1_0)
    z0, degp = _make_sc(True)(a0, idx_p)
    deg = degp[:, :, None]
    y1, a1 = _tc_mid(z0, deg, y, W2_0, W3_0, b3_0, W1_1, b1_1)
    (z1,) = _make_sc(False)(a1, idx_p)
    return _tc_final(z1, deg, y1, W2_1, W3_1, b3_1)


# speedup vs baseline: 14.7691x; 1.4575x over previous
import functools

import jax
import jax.numpy as jnp
from jax import lax
from jax.experimental import pallas as pl
from jax.experimental.pallas import tpu as pltpu
from jax.experimental.pallas import tpu_sc as plsc

N = 10000
E = 320000
D = 128
NC = 2
NS = 16
NW = NC * NS
EP = E // NW
K = 80
NCHUNK = EP // K
NPAD = 10240
ZROWS = NPAD // NS
DEGW = NPAD // NS
LG = D // 16

_GATHER_DN = lax.GatherDimensionNumbers(
    offset_dims=(), collapsed_slice_dims=(0,), start_index_map=(0,))


def _splat(v16, e):
    return lax.gather(v16, jnp.full((16, 1), e, jnp.int32), _GATHER_DN, (1,),
                      mode=lax.GatherScatterMode.PROMISE_IN_BOUNDS)


def _sc_body(compute_deg, a_hbm, idx_hbm, *rest):
    if compute_deg:
        (z_out, deg_out, zsh, dsh, ib0, ib1, ib2, ib3, ib4,
         rows0, rows1, rows2, rows3, ws0, ws1, ws2, ws3,
         ds0, ds1, ds2, ds3, zbuf, zd,
         i0, i1, i2, i3, i4, g0, g1, g2, g3, s0, s1, s2, s3,
         d0, d1, d2, d3, zsem) = rest
        dsem = (d0, d1, d2, d3)
    else:
        (z_out, zsh, ib0, ib1, ib2, ib3, ib4,
         rows0, rows1, rows2, rows3, ws0, ws1, ws2, ws3,
         ds0, ds1, ds2, ds3, zbuf,
         i0, i1, i2, i3, i4, g0, g1, g2, g3, s0, s1, s2, s3, zsem) = rest
    ib = (ib0, ib1, ib2, ib3, ib4)
    isem = (i0, i1, i2, i3, i4)
    rowsb = (rows0, rows1, rows2, rows3)
    wstage = (ws0, ws1, ws2, ws3)
    dstage = (ds0, ds1, ds2, ds3)
    gsem = (g0, g1, g2, g3)
    ssem = (s0, s1, s2, s3)
    c = lax.axis_index("c")
    s = lax.axis_index("s")
    wid = s * NC + c
    zero16 = jnp.zeros((16,), jnp.float32)

    pltpu.sync_copy(idx_hbm.at[wid, 0], ib0)
    pltpu.sync_copy(idx_hbm.at[wid, 1], ib1)
    pltpu.sync_copy(idx_hbm.at[wid, 2], ib2)
    pltpu.async_copy(a_hbm.at[ib0.at[0]], rows0, g0)
    pltpu.async_copy(a_hbm.at[ib1.at[0]], rows1, g1)

    def zero_zbuf(i, _):
        for j in range(LG):
            zbuf[i, pl.ds(j * 16, 16)] = zero16
        return 0
    lax.fori_loop(0, 16, zero_zbuf, 0)

    def zero_zsh(k, _):
        pltpu.async_copy(zbuf, zsh.at[pl.ds(s * ZROWS + k * 16, 16)], zsem)
        return 0
    lax.fori_loop(0, ZROWS // 16, zero_zsh, 0)

    if compute_deg:
        def zero_zd(i, _):
            zd[pl.ds(i * 16, 16)] = zero16
            return 0
        lax.fori_loop(0, DEGW // 16, zero_zd, 0)
        pltpu.async_copy(zd, dsh.at[pl.ds(s * DEGW, DEGW)], zsem)

    def drain_zero(k, _):
        pltpu.make_async_copy(zbuf, zsh.at[pl.ds(s * ZROWS, 16)], zsem).wait()
        return 0
    lax.fori_loop(0, ZROWS // 16, drain_zero, 0)
    if compute_deg:
        pltpu.make_async_copy(zd, dsh.at[pl.ds(s * DEGW, DEGW)], zsem).wait()

    plsc.subcore_barrier()

    def scale(t):
        rows = rowsb[t]
        ws = wstage[t]

        def grp(g, _):
            w16 = ws[pl.ds(g * 16, 16)]
            for e in range(16):
                we = _splat(w16, e)
                for j in range(LG):
                    rows[g * 16 + e, pl.ds(j * 16, 16)] = (
                        rows[g * 16 + e, pl.ds(j * 16, 16)] * we)
            return 0
        lax.fori_loop(0, K // 16, grp, 0)

    def chunk(i, _):
        for t4 in range(4):
            @pl.when(jnp.logical_and(i % 4 == t4, i >= 2))
            def _(t4=t4):
                tn = (t4 + 2) % 4
                pltpu.make_async_copy(rowsb[tn], zsh.at[ib0.at[1]],
                                      ssem[tn]).wait()
                if compute_deg:
                    pltpu.make_async_copy(ws0, dsh.at[ib0.at[1]],
                                          dsem[tn]).wait()

        for q5 in range(5):
            @pl.when(jnp.logical_and(i % 5 == q5, i + 3 < NCHUNK))
            def _(q5=q5):
                qp = (q5 + 3) % 5
                pltpu.async_copy(idx_hbm.at[wid, i + 3], ib[qp], isem[qp])

        for r in range(20):
            @pl.when(i % 20 == r)
            def _(r=r):
                qg = (r + 2) % 5
                tg = (r + 2) % 4

                @pl.when(jnp.logical_and(i + 2 < NCHUNK, i >= 1))
                def _():
                    pltpu.make_async_copy(idx_hbm.at[wid, 0], ib[qg],
                                          isem[qg]).wait()

                @pl.when(i + 2 < NCHUNK)
                def _():
                    pltpu.async_copy(a_hbm.at[ib[qg].at[0]], rowsb[tg],
                                     gsem[tg])

                q = r % 5
                t = r % 4
                for g in range(K // 16):
                    wstage[t][pl.ds(g * 16, 16)] = plsc.bitcast(
                        ib[q][2, pl.ds(g * 16, 16)], jnp.float32)
                    dstage[t][pl.ds(g * 16, 16)] = ib[q][1, pl.ds(g * 16, 16)]

        for t4 in range(4):
            @pl.when(i % 4 == t4)
            def _(t4=t4):
                pltpu.make_async_copy(a_hbm.at[ib0.at[0]], rowsb[t4],
                                      gsem[t4]).wait()
                scale(t4)
                pltpu.async_copy(rowsb[t4], zsh.at[dstage[t4]], ssem[t4],
                                 add=True)
                if compute_deg:
                    pltpu.async_copy(wstage[t4], dsh.at[dstage[t4]], dsem[t4],
                                     add=True)
        return 0
    lax.fori_loop(0, NCHUNK, chunk, 0)

    for lc in (NCHUNK - 2, NCHUNK - 1):
        lt = lc % 4
        pltpu.make_async_copy(rowsb[lt], zsh.at[ib0.at[1]], ssem[lt]).wait()
        if compute_deg:
            pltpu.make_async_copy(ws0, dsh.at[ib0.at[1]], dsem[lt]).wait()

    plsc.subcore_barrier()

    pltpu.sync_copy(zsh.at[pl.ds(s * ZROWS, ZROWS)],
                    z_out.at[c, pl.ds(s * ZROWS, ZROWS)])
    if compute_deg:
        pltpu.sync_copy(dsh.at[pl.ds(s * DEGW, DEGW)],
                        deg_out.at[c, pl.ds(s * DEGW, DEGW)])


@functools.cache
def _make_sc(compute_deg):
    mesh = plsc.VectorSubcoreMesh(core_axis_name="c", subcore_axis_name="s")
    out_type = [jax.ShapeDtypeStruct((NC, NPAD, D), jnp.float32)]
    scratch = [pltpu.VMEM_SHARED((NPAD, D), jnp.float32)]
    if compute_deg:
        out_type.append(jax.ShapeDtypeStruct((NC, NPAD), jnp.float32))
        scratch.append(pltpu.VMEM_SHARED((NPAD,), jnp.float32))
    scratch += [
        pltpu.VMEM((3, K), jnp.int32),
        pltpu.VMEM((3, K), jnp.int32),
        pltpu.VMEM((3, K), jnp.int32),
        pltpu.VMEM((3, K), jnp.int32),
        pltpu.VMEM((3, K), jnp.int32),
        pltpu.VMEM((K, D), jnp.float32),
        pltpu.VMEM((K, D), jnp.float32),
        pltpu.VMEM((K, D), jnp.float32),
        pltpu.VMEM((K, D), jnp.float32),
        pltpu.VMEM((K,), jnp.float32),
        pltpu.VMEM((K,), jnp.float32),
        pltpu.VMEM((K,), jnp.float32),
        pltpu.VMEM((K,), jnp.float32),
        pltpu.VMEM((K,), jnp.int32),
        pltpu.VMEM((K,), jnp.int32),
        pltpu.VMEM((K,), jnp.int32),
        pltpu.VMEM((K,), jnp.int32),
        pltpu.VMEM((16, D), jnp.float32),
    ]
    if compute_deg:
        scratch.append(pltpu.VMEM((DEGW,), jnp.float32))
    nsem = 18 if compute_deg else 14
    scratch += [pltpu.SemaphoreType.DMA] * nsem
    return pl.kernel(
        functools.partial(_sc_body, compute_deg),
        out_type=out_type,
        mesh=mesh,
        scratch_types=scratch,
        compiler_params=pltpu.CompilerParams(needs_layout_passes=False),
    )


def _tc_lin_body(y_ref, w_ref, b_ref, o_ref):
    o_ref[...] = jnp.dot(y_ref[...], w_ref[...],
                         preferred_element_type=jnp.float32) + b_ref[...]


def _tc_lin(y, W1, b1):
    B = 2000
    return pl.pallas_call(
        _tc_lin_body,
        grid=(N // B,),
        in_specs=[pl.BlockSpec((B, D), lambda i: (i, 0)),
                  pl.BlockSpec((D, D), lambda i: (0, 0)),
                  pl.BlockSpec((1, D), lambda i: (0, 0))],
        out_specs=pl.BlockSpec((B, D), lambda i: (i, 0)),
        out_shape=jax.ShapeDtypeStruct((N, D), jnp.float32),
    )(y, W1, b1.reshape(1, D))


def _combine(z_ref, deg_ref, y_ref, w2_ref, w3_ref, b3_ref):
    yv = y_ref[...]
    z = z_ref[0] + z_ref[1]
    deg = deg_ref[0] + deg_ref[1]
    t = (z - deg * jnp.dot(yv, w2_ref[...], preferred_element_type=jnp.float32)
         + jnp.dot(yv, w3_ref[...], preferred_element_type=jnp.float32)
         + b3_ref[...])
    return jnp.where(t >= 0, t, 0.01 * t)


def _tc_mid_body(z_ref, deg_ref, y_ref, w2_ref, w3_ref, b3_ref, w1n_ref,
                 b1n_ref, y1_ref, a1_ref):
    y1 = _combine(z_ref, deg_ref, y_ref, w2_ref, w3_ref, b3_ref)
    y1_ref[...] = y1
    a1_ref[...] = jnp.dot(y1, w1n_ref[...],
                          preferred_element_type=jnp.float32) + b1n_ref[...]


def _tc_mid(z, deg, y, W2, W3, b3, W1n, b1n):
    B = 2000
    return pl.pallas_call(
        _tc_mid_body,
        grid=(N // B,),
        in_specs=[pl.BlockSpec((NC, B, D), lambda i: (0, i, 0)),
                  pl.BlockSpec((NC, B, 1), lambda i: (0, i, 0)),
                  pl.BlockSpec((B, D), lambda i: (i, 0)),
                  pl.BlockSpec((D, D), lambda i: (0, 0)),
                  pl.BlockSpec((D, D), lambda i: (0, 0)),
                  pl.BlockSpec((1, D), lambda i: (0, 0)),
                  pl.BlockSpec((D, D), lambda i: (0, 0)),
                  pl.BlockSpec((1, D), lambda i: (0, 0))],
        out_specs=[pl.BlockSpec((B, D), lambda i: (i, 0)),
                   pl.BlockSpec((B, D), lambda i: (i, 0))],
        out_shape=[jax.ShapeDtypeStruct((N, D), jnp.float32),
                   jax.ShapeDtypeStruct((N, D), jnp.float32)],
    )(z, deg, y, W2, W3, b3.reshape(1, D), W1n, b1n.reshape(1, D))


def _tc_final_body(z_ref, deg_ref, y_ref, w2_ref, w3_ref, b3_ref, o_ref):
    o_ref[...] = _combine(z_ref, deg_ref, y_ref, w2_ref, w3_ref, b3_ref)


def _tc_final(z, deg, y, W2, W3, b3):
    B = 2000
    return pl.pallas_call(
        _tc_final_body,
        grid=(N // B,),
        in_specs=[pl.BlockSpec((NC, B, D), lambda i: (0, i, 0)),
                  pl.BlockSpec((NC, B, 1), lambda i: (0, i, 0)),
                  pl.BlockSpec((B, D), lambda i: (i, 0)),
                  pl.BlockSpec((D, D), lambda i: (0, 0)),
                  pl.BlockSpec((D, D), lambda i: (0, 0)),
                  pl.BlockSpec((1, D), lambda i: (0, 0))],
        out_specs=pl.BlockSpec((B, D), lambda i: (i, 0)),
        out_shape=jax.ShapeDtypeStruct((N, D), jnp.float32),
    )(z, deg, y, W2, W3, b3.reshape(1, D))


def kernel(y, edge_index, edge_weight,
           W1_0, b1_0, W2_0, W3_0, b3_0,
           W1_1, b1_1, W2_1, W3_1, b3_1):
    w_bits = lax.bitcast_convert_type(edge_weight, jnp.int32)
    idx_p = jnp.stack(
        [edge_index[0].reshape(NW, NCHUNK, K),
         edge_index[1].reshape(NW, NCHUNK, K),
         w_bits.reshape(NW, NCHUNK, K)], axis=2)
    a0 = _tc_lin(y, W1_0, b1_0)
    z0, degp = _make_sc(True)(a0, idx_p)
    deg = degp[:, :, None]
    y1, a1 = _tc_mid(z0, deg, y, W2_0, W3_0, b3_0, W1_1, b1_1)
    (z1,) = _make_sc(False)(a1, idx_p)
    return _tc_final(z1, deg, y1, W2_1, W3_1, b3_1)
